# Initial kernel scaffold; baseline (speedup 1.0000x reference)
import jax
import jax.numpy as jnp
from jax.experimental import pallas as pl


def kernel(x, edge_index, batch, node_emb, comp_emb, pin_emb, proj_W, proj_b, gin_W1, gin_b1, gin_W2, gin_b2, bn_g, bn_b, fus_W1, fus_b1, fus_W2, fus_b2, clf_W, clf_b):
    raise NotImplementedError("write your pallas kernel here")


# speedup vs baseline: 2.7440x; 2.7440x over previous
import functools

import jax
import jax.numpy as jnp
from jax import lax
from jax.experimental import pallas as pl
from jax.experimental.pallas import tpu as pltpu
from jax.experimental.pallas import tpu_sc as plsc

H = 128
G = 64
NC = 2
NS = 16
NW = NC * NS
EC = 80
ZB = 80


def _edge_agg_sc(h, src2, dst2, n_pad):
    nch = src2.shape[0] // NW
    rows_w = n_pad // NS
    nzb = rows_w // ZB
    mesh = plsc.VectorSubcoreMesh(core_axis_name="c", subcore_axis_name="s")

    @functools.partial(
        pl.kernel,
        out_type=jax.ShapeDtypeStruct((NC, n_pad, H), jnp.float32),
        mesh=mesh,
        scratch_types=[
            pltpu.VMEM_SHARED((n_pad, H), jnp.float32),
            pltpu.VMEM((nch, EC), jnp.int32),
            pltpu.VMEM((nch, EC), jnp.int32),
            pltpu.VMEM((EC, H), jnp.float32),
            pltpu.SemaphoreType.DMA,
        ],
    )
    def k(h_hbm, src_hbm, dst_hbm, out_hbm, acc, sidx, didx, rows, sem):
        c = lax.axis_index("c")
        s = lax.axis_index("s")
        wid = c * NS + s

        zero = jnp.zeros((16,), jnp.float32)

        def zrow(i, carry):
            for j in range(H // 16):
                rows[i, pl.ds(j * 16, 16)] = zero
            return carry

        lax.fori_loop(0, EC, zrow, 0)
        base = s * rows_w
        for kk in range(nzb):
            pltpu.sync_copy(rows, acc.at[pl.ds(base + kk * ZB, ZB)])
        plsc.subcore_barrier()

        pltpu.sync_copy(src_hbm.at[pl.ds(wid * nch, nch)], sidx)
        pltpu.sync_copy(dst_hbm.at[pl.ds(wid * nch, nch)], didx)

        def body(j, carry):
            pltpu.async_copy(h_hbm.at[sidx.at[j]], rows, sem).wait()
            pltpu.sync_copy(rows, acc.at[didx.at[j]], add=True)
            return carry

        lax.fori_loop(0, nch, body, 0)
        plsc.subcore_barrier()

        for kk in range(nzb):
            sl = pl.ds(base + kk * ZB, ZB)
            pltpu.sync_copy(acc.at[sl], rows)
            pltpu.sync_copy(rows, out_hbm.at[c].at[sl])

    return k(h, src2, dst2)


def _embed_tc(nt, ct, pt, node_emb, comp_emb, pin_emb, proj_W, proj_b):
    n = nt.shape[0]
    blk = 1000
    nb = n // blk

    def body(nt_r, ct_r, pt_r, ne_r, ce_r, pe_r, w_r, b_r, o_r):
        f32 = jnp.float32
        tab = jnp.concatenate([
            jnp.dot(ne_r[...], w_r[0:H, :], preferred_element_type=f32),
            jnp.dot(ce_r[...], w_r[H:2 * H, :], preferred_element_type=f32),
            jnp.dot(pe_r[...], w_r[2 * H:3 * H, :], preferred_element_type=f32),
        ], axis=0)
        oh = jnp.concatenate([
            (nt_r[...] == lax.broadcasted_iota(jnp.int32, (blk, 5), 1)).astype(f32),
            (ct_r[...] == lax.broadcasted_iota(jnp.int32, (blk, 6), 1)).astype(f32),
            (pt_r[...] == lax.broadcasted_iota(jnp.int32, (blk, 6), 1)).astype(f32),
        ], axis=1)
        o_r[...] = jnp.dot(oh, tab, preferred_element_type=f32) + b_r[...]

    col = pl.BlockSpec((blk, 1), lambda b: (b, 0))
    full = lambda a: pl.BlockSpec(a.shape, lambda b: tuple(0 for _ in a.shape))
    return pl.pallas_call(
        body,
        grid=(nb,),
        in_specs=[col, col, col, full(node_emb), full(comp_emb),
                  full(pin_emb), full(proj_W), pl.BlockSpec((1, H), lambda b: (0, 0))],
        out_specs=pl.BlockSpec((blk, H), lambda b: (b, 0)),
        out_shape=jax.ShapeDtypeStruct((n, H), jnp.float32),
    )(nt, ct, pt, node_emb, comp_emb, pin_emb, proj_W, proj_b.reshape(1, H))


def _gin_tc(h, parts, w1, b1, w2, b2, g, bb, residual):
    n = h.shape[0]
    blk = 1000
    nb = n // blk
    f32 = jnp.float32

    def body(h_r, p_r, w1_r, b1_r, w2_r, b2_r, g_r, bb_r, o_r, vbuf, ssum, ssq):
        ph = pl.program_id(0)
        b = pl.program_id(1)

        @pl.when(jnp.logical_and(ph == 0, b == 0))
        def _():
            ssum[...] = jnp.zeros_like(ssum)
            ssq[...] = jnp.zeros_like(ssq)

        @pl.when(ph == 0)
        def _():
            z = h_r[...] + p_r[0] + p_r[1]
            u = jnp.maximum(
                jnp.dot(z, w1_r[...], preferred_element_type=f32) + b1_r[...], 0.0)
            v = jnp.dot(u, w2_r[...], preferred_element_type=f32) + b2_r[...]
            vbuf[pl.ds(b * blk, blk), :] = v
            ssum[...] += jnp.sum(v, axis=0, keepdims=True)
            ssq[...] += jnp.sum(v * v, axis=0, keepdims=True)

        @pl.when(ph == 1)
        def _():
            mean = ssum[...] * (1.0 / n)
            var = ssq[...] * (1.0 / n) - mean * mean
            inv = lax.rsqrt(var + 1e-5)
            v = vbuf[pl.ds(b * blk, blk), :]
            zz = jnp.maximum((v - mean) * inv * g_r[...] + bb_r[...], 0.0)
            if residual:
                zz = zz + h_r[...]
            o_r[...] = zz

    rowblk = pl.BlockSpec((blk, H), lambda p, b: (b, 0))
    full = lambda a: pl.BlockSpec(a.shape, lambda p, b: tuple(0 for _ in a.shape))
    return pl.pallas_call(
        body,
        grid=(2, nb),
        in_specs=[rowblk, pl.BlockSpec((NC, blk, H), lambda p, b: (0, b, 0)),
                  full(w1), pl.BlockSpec((1, 2 * H), lambda p, b: (0, 0)),
                  full(w2), pl.BlockSpec((1, H), lambda p, b: (0, 0)),
                  pl.BlockSpec((1, H), lambda p, b: (0, 0)),
                  pl.BlockSpec((1, H), lambda p, b: (0, 0))],
        out_specs=rowblk,
        out_shape=jax.ShapeDtypeStruct((n, H), f32),
        scratch_shapes=[pltpu.VMEM((n, H), f32), pltpu.VMEM((1, H), f32),
                        pltpu.VMEM((1, H), f32)],
    )(h, parts, w1, b1.reshape(1, 2 * H), w2, b2.reshape(1, H),
      g.reshape(1, H), bb.reshape(1, H))


def _pool_fuse_tc(h, batch, fw1, fb1, fw2, fb2, cw, cb):
    n = h.shape[0]
    f32 = jnp.float32

    def body(h_r, bat_r, fw1_r, fb1_r, fw2_r, fb2_r, cw_r, cb_r, o_r, gmax):
        hv = h_r[...]
        bat = bat_r[...]
        mask = (bat == lax.broadcasted_iota(jnp.int32, (n, G), 1)).astype(f32)
        gsum = lax.dot_general(mask, hv, (((0,), (0,)), ((), ())),
                               preferred_element_type=f32)
        cnt = jnp.sum(mask, axis=0)[:, None]

        def mx(gg, carry):
            m = jnp.max(jnp.where(bat == gg, hv, -jnp.inf), axis=0, keepdims=True)
            gmax[pl.ds(gg, 1), :] = m
            return carry

        lax.fori_loop(0, G, mx, 0)
        gmean = gsum / jnp.maximum(cnt, 1.0)
        emb = jnp.concatenate([gmean, gmax[...], gsum], axis=1)
        f1 = jnp.maximum(
            jnp.dot(emb, fw1_r[...], preferred_element_type=f32) + fb1_r[...], 0.0)
        f2 = jnp.maximum(
            jnp.dot(f1, fw2_r[...], preferred_element_type=f32) + fb2_r[...], 0.0)
        o_r[...] = jnp.dot(f2, cw_r[...], preferred_element_type=f32) + cb_r[...]

    full = lambda a: pl.BlockSpec(a.shape, lambda: tuple(0 for _ in a.shape))
    args = (h, batch, fw1, fb1.reshape(1, 2 * H), fw2, fb2.reshape(1, H),
            cw, cb.reshape(1, 4))
    return pl.pallas_call(
        body,
        in_specs=[full(a) for a in args],
        out_specs=full(jnp.zeros((G, 4))),
        out_shape=jax.ShapeDtypeStruct((G, 4), f32),
        scratch_shapes=[pltpu.VMEM((G, H), f32)],
    )(*args)


def kernel(x, edge_index, batch, node_emb, comp_emb, pin_emb, proj_W, proj_b,
           gin_W1, gin_b1, gin_W2, gin_b2, bn_g, bn_b,
           fus_W1, fus_b1, fus_W2, fus_b2, clf_W, clf_b):
    n = x.shape[0]
    xi = jnp.clip(x, 0, None).astype(jnp.int32)
    nt, ct, pt = xi[:, 0:1], xi[:, 1:2], xi[:, 2:3]
    e = edge_index.shape[1]
    nch = ((-(-e // (NW * EC))) + 7) // 8 * 8
    e_pad = NW * nch * EC
    rows_w = (-(-n // (NS * ZB))) * ZB
    n_pad = NS * rows_w
    if n_pad <= n:
        n_pad += ZB * NS
    src = edge_index[0].astype(jnp.int32)
    dst = edge_index[1].astype(jnp.int32)
    src2 = jnp.concatenate(
        [src, jnp.zeros((e_pad - e,), jnp.int32)]).reshape(-1, EC)
    dst2 = jnp.concatenate(
        [dst, jnp.full((e_pad - e,), n, jnp.int32)]).reshape(-1, EC)

    h = _embed_tc(nt, ct, pt, node_emb, comp_emb, pin_emb, proj_W, proj_b)
    for i in range(len(gin_W1)):
        parts = _edge_agg_sc(h, src2, dst2, n_pad)[:, :n]
        h = _gin_tc(h, parts, gin_W1[i], gin_b1[i], gin_W2[i], gin_b2[i],
                    bn_g[i], bn_b[i], residual=(i > 0))
    return _pool_fuse_tc(h, batch.reshape(n, 1).astype(jnp.int32),
                         fus_W1, fus_b1, fus_W2, fus_b2, clf_W, clf_b)


# speedup vs baseline: 5.5220x; 2.0124x over previous
import functools

import jax
import jax.numpy as jnp
from jax import lax
from jax.experimental import pallas as pl
from jax.experimental.pallas import tpu as pltpu
from jax.experimental.pallas import tpu_sc as plsc

H = 128
G = 64
NC = 2
NS = 16
NW = NC * NS
EC = 80
ZB = 80


def _edge_agg_sc(h, src2, dst2, n_pad):
    nch = src2.shape[0] // NW
    rows_w = n_pad // NS
    nzb = rows_w // ZB
    mesh = plsc.VectorSubcoreMesh(core_axis_name="c", subcore_axis_name="s")

    @functools.partial(
        pl.kernel,
        out_type=jax.ShapeDtypeStruct((NC, n_pad, H), jnp.float32),
        mesh=mesh,
        scratch_types=[
            pltpu.VMEM_SHARED((n_pad, H), jnp.float32),
            pltpu.VMEM((nch, EC), jnp.int32),
            pltpu.VMEM((nch, EC), jnp.int32),
            pltpu.VMEM((EC, H), jnp.float32),
            pltpu.SemaphoreType.DMA,
        ],
    )
    def k(h_hbm, src_hbm, dst_hbm, out_hbm, acc, sidx, didx, rows, sem):
        c = lax.axis_index("c")
        s = lax.axis_index("s")
        wid = c * NS + s

        zero = jnp.zeros((16,), jnp.float32)

        def zrow(i, carry):
            for j in range(H // 16):
                rows[i, pl.ds(j * 16, 16)] = zero
            return carry

        lax.fori_loop(0, EC, zrow, 0)
        base = s * rows_w
        for kk in range(nzb):
            pltpu.sync_copy(rows, acc.at[pl.ds(base + kk * ZB, ZB)])
        plsc.subcore_barrier()

        pltpu.sync_copy(src_hbm.at[pl.ds(wid * nch, nch)], sidx)
        pltpu.sync_copy(dst_hbm.at[pl.ds(wid * nch, nch)], didx)

        def body(j, carry):
            pltpu.async_copy(h_hbm.at[sidx.at[j]], rows, sem).wait()
            pltpu.sync_copy(rows, acc.at[didx.at[j]], add=True)
            return carry

        lax.fori_loop(0, nch, body, 0)
        plsc.subcore_barrier()

        for kk in range(nzb):
            sl = pl.ds(base + kk * ZB, ZB)
            pltpu.sync_copy(acc.at[sl], rows)
            pltpu.sync_copy(rows, out_hbm.at[c].at[sl])

    return k(h, src2, dst2)


def _embed_tc(nt, ct, pt, node_emb, comp_emb, pin_emb, proj_W, proj_b):
    n = nt.shape[0]
    blk = 1000
    nb = n // blk

    def body(nt_r, ct_r, pt_r, ne_r, ce_r, pe_r, w_r, b_r, o_r):
        f32 = jnp.float32
        tab = jnp.concatenate([
            jnp.dot(ne_r[...], w_r[0:H, :], preferred_element_type=f32),
            jnp.dot(ce_r[...], w_r[H:2 * H, :], preferred_element_type=f32),
            jnp.dot(pe_r[...], w_r[2 * H:3 * H, :], preferred_element_type=f32),
        ], axis=0)
        oh = jnp.concatenate([
            (nt_r[...] == lax.broadcasted_iota(jnp.int32, (blk, 5), 1)).astype(f32),
            (ct_r[...] == lax.broadcasted_iota(jnp.int32, (blk, 6), 1)).astype(f32),
            (pt_r[...] == lax.broadcasted_iota(jnp.int32, (blk, 6), 1)).astype(f32),
        ], axis=1)
        o_r[...] = jnp.dot(oh, tab, preferred_element_type=f32) + b_r[...]

    col = pl.BlockSpec((blk, 1), lambda b: (b, 0))
    full = lambda a: pl.BlockSpec(a.shape, lambda b: tuple(0 for _ in a.shape))
    return pl.pallas_call(
        body,
        grid=(nb,),
        in_specs=[col, col, col, full(node_emb), full(comp_emb),
                  full(pin_emb), full(proj_W), pl.BlockSpec((1, H), lambda b: (0, 0))],
        out_specs=pl.BlockSpec((blk, H), lambda b: (b, 0)),
        out_shape=jax.ShapeDtypeStruct((n, H), jnp.float32),
    )(nt, ct, pt, node_emb, comp_emb, pin_emb, proj_W, proj_b.reshape(1, H))


def _gin_tc(h, parts, w1, b1, w2, b2, g, bb, residual):
    n = h.shape[0]
    blk = 1000
    nb = n // blk
    f32 = jnp.float32

    def body(h_r, p_r, w1_r, b1_r, w2_r, b2_r, g_r, bb_r, o_r, vbuf, ssum, ssq):
        ph = pl.program_id(0)
        b = pl.program_id(1)

        @pl.when(jnp.logical_and(ph == 0, b == 0))
        def _():
            ssum[...] = jnp.zeros_like(ssum)
            ssq[...] = jnp.zeros_like(ssq)

        @pl.when(ph == 0)
        def _():
            z = h_r[...] + p_r[0] + p_r[1]
            u = jnp.maximum(
                jnp.dot(z, w1_r[...], preferred_element_type=f32) + b1_r[...], 0.0)
            v = jnp.dot(u, w2_r[...], preferred_element_type=f32) + b2_r[...]
            vbuf[pl.ds(b * blk, blk), :] = v
            ssum[...] += jnp.sum(v, axis=0, keepdims=True)
            ssq[...] += jnp.sum(v * v, axis=0, keepdims=True)

        @pl.when(ph == 1)
        def _():
            mean = ssum[...] * (1.0 / n)
            var = ssq[...] * (1.0 / n) - mean * mean
            inv = lax.rsqrt(var + 1e-5)
            v = vbuf[pl.ds(b * blk, blk), :]
            zz = jnp.maximum((v - mean) * inv * g_r[...] + bb_r[...], 0.0)
            if residual:
                zz = zz + h_r[...]
            o_r[...] = zz

    rowblk = pl.BlockSpec((blk, H), lambda p, b: (b, 0))
    full = lambda a: pl.BlockSpec(a.shape, lambda p, b: tuple(0 for _ in a.shape))
    return pl.pallas_call(
        body,
        grid=(2, nb),
        in_specs=[rowblk, pl.BlockSpec((NC, blk, H), lambda p, b: (0, b, 0)),
                  full(w1), pl.BlockSpec((1, 2 * H), lambda p, b: (0, 0)),
                  full(w2), pl.BlockSpec((1, H), lambda p, b: (0, 0)),
                  pl.BlockSpec((1, H), lambda p, b: (0, 0)),
                  pl.BlockSpec((1, H), lambda p, b: (0, 0))],
        out_specs=rowblk,
        out_shape=jax.ShapeDtypeStruct((n, H), f32),
        scratch_shapes=[pltpu.VMEM((n, H), f32), pltpu.VMEM((1, H), f32),
                        pltpu.VMEM((1, H), f32)],
    )(h, parts, w1, b1.reshape(1, 2 * H), w2, b2.reshape(1, H),
      g.reshape(1, H), bb.reshape(1, H))


def _pool_fuse_tc(h, batch, fw1, fb1, fw2, fb2, cw, cb):
    n = h.shape[0]
    f32 = jnp.float32

    def body(h_r, bat_r, fw1_r, fb1_r, fw2_r, fb2_r, cw_r, cb_r, o_r, gmax):
        hv = h_r[...]
        bat = bat_r[...]
        mask = (bat == lax.broadcasted_iota(jnp.int32, (n, G), 1)).astype(f32)
        gsum = lax.dot_general(mask, hv, (((0,), (0,)), ((), ())),
                               preferred_element_type=f32)
        cnt = jnp.sum(mask, axis=0)[:, None]

        def mx(gg, carry):
            m = jnp.max(jnp.where(bat == gg, hv, -jnp.inf), axis=0, keepdims=True)
            gmax[pl.ds(gg, 1), :] = m
            return carry

        lax.fori_loop(0, G, mx, 0)
        gmean = gsum / jnp.maximum(cnt, 1.0)
        emb = jnp.concatenate([gmean, gmax[...], gsum], axis=1)
        f1 = jnp.maximum(
            jnp.dot(emb, fw1_r[...], preferred_element_type=f32) + fb1_r[...], 0.0)
        f2 = jnp.maximum(
            jnp.dot(f1, fw2_r[...], preferred_element_type=f32) + fb2_r[...], 0.0)
        o_r[...] = jnp.dot(f2, cw_r[...], preferred_element_type=f32) + cb_r[...]

    full = lambda a: pl.BlockSpec(a.shape, lambda: tuple(0 for _ in a.shape))
    args = (h, batch, fw1, fb1.reshape(1, 2 * H), fw2, fb2.reshape(1, H),
            cw, cb.reshape(1, 4))
    return pl.pallas_call(
        body,
        in_specs=[full(a) for a in args],
        out_specs=full(jnp.zeros((G, 4))),
        out_shape=jax.ShapeDtypeStruct((G, 4), f32),
        scratch_shapes=[pltpu.VMEM((G, H), f32)],
    )(*args)


def kernel(x, edge_index, batch, node_emb, comp_emb, pin_emb, proj_W, proj_b,
           gin_W1, gin_b1, gin_W2, gin_b2, bn_g, bn_b,
           fus_W1, fus_b1, fus_W2, fus_b2, clf_W, clf_b):
    n = x.shape[0]
    xi = jnp.clip(x, 0, None).astype(jnp.int32)
    nt, ct, pt = xi[:, 0:1], xi[:, 1:2], xi[:, 2:3]
    e = edge_index.shape[1]
    nch = ((-(-e // (NW * EC))) + 7) // 8 * 8
    e_pad = NW * nch * EC
    rows_w = (-(-n // (NS * ZB))) * ZB
    n_pad = NS * rows_w
    if n_pad <= n:
        n_pad += ZB * NS
    src = edge_index[0].astype(jnp.int32)
    dst = edge_index[1].astype(jnp.int32)
    pad_ar = jnp.arange(e_pad - e, dtype=jnp.int32)
    src2 = jnp.concatenate([src, pad_ar % n]).reshape(-1, EC)
    dst2 = jnp.concatenate(
        [dst, n + pad_ar % (n_pad - n)]).reshape(-1, EC)

    h = _embed_tc(nt, ct, pt, node_emb, comp_emb, pin_emb, proj_W, proj_b)
    for i in range(len(gin_W1)):
        parts = _edge_agg_sc(h, src2, dst2, n_pad)[:, :n]
        h = _gin_tc(h, parts, gin_W1[i], gin_b1[i], gin_W2[i], gin_b2[i],
                    bn_g[i], bn_b[i], residual=(i > 0))
    return _pool_fuse_tc(h, batch.reshape(n, 1).astype(jnp.int32),
                         fus_W1, fus_b1, fus_W2, fus_b2, clf_W, clf_b)


# speedup vs baseline: 8.4659x; 1.5331x over previous
import functools

import jax
import jax.numpy as jnp
from jax import lax
from jax.experimental import pallas as pl
from jax.experimental.pallas import tpu as pltpu
from jax.experimental.pallas import tpu_sc as plsc

H = 128
G = 64
NC = 2
NS = 16
NW = NC * NS
EC = 80
ZB = 80


def _edge_agg_sc(h, comb3, n_pad):
    nch = comb3.shape[1]
    rows_w = n_pad // NS
    nzb = rows_w // ZB
    mesh = plsc.VectorSubcoreMesh(core_axis_name="c", subcore_axis_name="s")

    @functools.partial(
        pl.kernel,
        out_type=jax.ShapeDtypeStruct((NC, n_pad, H), jnp.float32),
        mesh=mesh,
        scratch_types=[
            pltpu.VMEM_SHARED((n_pad, H), jnp.float32),
            pltpu.VMEM((nch, EC), jnp.int32),
            pltpu.VMEM((EC,), jnp.int32),
            pltpu.VMEM((EC,), jnp.int32),
            pltpu.VMEM((EC,), jnp.int32),
            pltpu.VMEM((EC,), jnp.int32),
            pltpu.VMEM((EC, H), jnp.float32),
            pltpu.VMEM((EC, H), jnp.float32),
            pltpu.SemaphoreType.DMA,
            pltpu.SemaphoreType.DMA,
        ],
    )
    def k(h_hbm, comb_hbm, out_hbm, acc, comb, sb0, sb1, db0, db1,
          r0, r1, sem0, sem1):
        c = lax.axis_index("c")
        s = lax.axis_index("s")
        wid = c * NS + s

        icpy = pltpu.async_copy(comb_hbm.at[wid], comb, sem0)
        zero = jnp.zeros((16,), jnp.float32)

        def zrow(i, carry):
            for j in range(H // 16):
                r0[i, pl.ds(j * 16, 16)] = zero
            return carry

        lax.fori_loop(0, EC, zrow, 0)
        base = s * rows_w
        for kk in range(nzb):
            pltpu.sync_copy(r0, acc.at[pl.ds(base + kk * ZB, ZB)])
        icpy.wait()
        plsc.subcore_barrier()

        def prep(j, sb, db):
            for t in range(EC // 16):
                cw = comb[j, pl.ds(t * 16, 16)]
                sb[pl.ds(t * 16, 16)] = cw & 0xFFFF
                db[pl.ds(t * 16, 16)] = cw >> 16

        def gather(sb, r, sem):
            return pltpu.async_copy(h_hbm.at[sb], r, sem)

        def wait(sb, r, sem):
            pltpu.make_async_copy(h_hbm.at[sb], r, sem).wait()

        def scatter(r, db):
            pltpu.sync_copy(r, acc.at[db], add=True)

        prep(0, sb0, db0)
        gather(sb0, r0, sem0)

        def pair(i, carry):
            j = 2 * i
            prep(j + 1, sb1, db1)
            gather(sb1, r1, sem1)
            wait(sb0, r0, sem0)
            scatter(r0, db0)

            @pl.when(j + 2 < nch)
            def _():
                prep(j + 2, sb0, db0)
                gather(sb0, r0, sem0)

            wait(sb1, r1, sem1)
            scatter(r1, db1)
            return carry

        lax.fori_loop(0, (nch - 1) // 2, pair, 0)
        if nch % 2 == 0:
            prep(nch - 1, sb1, db1)
            gather(sb1, r1, sem1)
            wait(sb0, r0, sem0)
            scatter(r0, db0)
            wait(sb1, r1, sem1)
            scatter(r1, db1)
        else:
            wait(sb0, r0, sem0)
            scatter(r0, db0)
        plsc.subcore_barrier()

        for kk in range(nzb):
            sl = pl.ds(base + kk * ZB, ZB)
            pltpu.sync_copy(acc.at[sl], r0)
            pltpu.sync_copy(r0, out_hbm.at[c].at[sl])

    return k(h, comb3)


def _embed_tc(nt, ct, pt, node_emb, comp_emb, pin_emb, proj_W, proj_b):
    n = nt.shape[0]
    blk = 1000
    nb = n // blk

    def body(nt_r, ct_r, pt_r, ne_r, ce_r, pe_r, w_r, b_r, o_r):
        f32 = jnp.float32
        tab = jnp.concatenate([
            jnp.dot(ne_r[...], w_r[0:H, :], preferred_element_type=f32),
            jnp.dot(ce_r[...], w_r[H:2 * H, :], preferred_element_type=f32),
            jnp.dot(pe_r[...], w_r[2 * H:3 * H, :], preferred_element_type=f32),
        ], axis=0)
        oh = jnp.concatenate([
            (nt_r[...] == lax.broadcasted_iota(jnp.int32, (blk, 5), 1)).astype(f32),
            (ct_r[...] == lax.broadcasted_iota(jnp.int32, (blk, 6), 1)).astype(f32),
            (pt_r[...] == lax.broadcasted_iota(jnp.int32, (blk, 6), 1)).astype(f32),
        ], axis=1)
        o_r[...] = jnp.dot(oh, tab, preferred_element_type=f32) + b_r[...]

    col = pl.BlockSpec((blk, 1), lambda b: (b, 0))
    full = lambda a: pl.BlockSpec(a.shape, lambda b: tuple(0 for _ in a.shape))
    return pl.pallas_call(
        body,
        grid=(nb,),
        in_specs=[col, col, col, full(node_emb), full(comp_emb),
                  full(pin_emb), full(proj_W), pl.BlockSpec((1, H), lambda b: (0, 0))],
        out_specs=pl.BlockSpec((blk, H), lambda b: (b, 0)),
        out_shape=jax.ShapeDtypeStruct((n, H), jnp.float32),
    )(nt, ct, pt, node_emb, comp_emb, pin_emb, proj_W, proj_b.reshape(1, H))


def _gin_tc(h, parts, w1, b1, w2, b2, g, bb, residual):
    n = h.shape[0]
    blk = 1000
    nb = n // blk
    f32 = jnp.float32

    def body(h_r, p_r, w1_r, b1_r, w2_r, b2_r, g_r, bb_r, o_r, vbuf, ssum, ssq):
        ph = pl.program_id(0)
        b = pl.program_id(1)

        @pl.when(jnp.logical_and(ph == 0, b == 0))
        def _():
            ssum[...] = jnp.zeros_like(ssum)
            ssq[...] = jnp.zeros_like(ssq)

        @pl.when(ph == 0)
        def _():
            z = h_r[...] + p_r[0] + p_r[1]
            u = jnp.maximum(
                jnp.dot(z, w1_r[...], preferred_element_type=f32) + b1_r[...], 0.0)
            v = jnp.dot(u, w2_r[...], preferred_element_type=f32) + b2_r[...]
            vbuf[pl.ds(b * blk, blk), :] = v
            ssum[...] += jnp.sum(v, axis=0, keepdims=True)
            ssq[...] += jnp.sum(v * v, axis=0, keepdims=True)

        @pl.when(ph == 1)
        def _():
            mean = ssum[...] * (1.0 / n)
            var = ssq[...] * (1.0 / n) - mean * mean
            inv = lax.rsqrt(var + 1e-5)
            v = vbuf[pl.ds(b * blk, blk), :]
            zz = jnp.maximum((v - mean) * inv * g_r[...] + bb_r[...], 0.0)
            if residual:
                zz = zz + h_r[...]
            o_r[...] = zz

    rowblk = pl.BlockSpec((blk, H), lambda p, b: (b, 0))
    full = lambda a: pl.BlockSpec(a.shape, lambda p, b: tuple(0 for _ in a.shape))
    return pl.pallas_call(
        body,
        grid=(2, nb),
        in_specs=[rowblk, pl.BlockSpec((NC, blk, H), lambda p, b: (0, b, 0)),
                  full(w1), pl.BlockSpec((1, 2 * H), lambda p, b: (0, 0)),
                  full(w2), pl.BlockSpec((1, H), lambda p, b: (0, 0)),
                  pl.BlockSpec((1, H), lambda p, b: (0, 0)),
                  pl.BlockSpec((1, H), lambda p, b: (0, 0))],
        out_specs=rowblk,
        out_shape=jax.ShapeDtypeStruct((n, H), f32),
        scratch_shapes=[pltpu.VMEM((n, H), f32), pltpu.VMEM((1, H), f32),
                        pltpu.VMEM((1, H), f32)],
    )(h, parts, w1, b1.reshape(1, 2 * H), w2, b2.reshape(1, H),
      g.reshape(1, H), bb.reshape(1, H))


def _pool_fuse_tc(h, batch, fw1, fb1, fw2, fb2, cw, cb):
    n = h.shape[0]
    f32 = jnp.float32

    def body(h_r, bat_r, fw1_r, fb1_r, fw2_r, fb2_r, cw_r, cb_r, o_r, gmax):
        hv = h_r[...]
        bat = bat_r[...]
        mask = (bat == lax.broadcasted_iota(jnp.int32, (n, G), 1)).astype(f32)
        gsum = lax.dot_general(mask, hv, (((0,), (0,)), ((), ())),
                               preferred_element_type=f32)
        cnt = jnp.sum(mask, axis=0)[:, None]

        def mx(gg, carry):
            m = jnp.max(jnp.where(bat == gg, hv, -jnp.inf), axis=0, keepdims=True)
            gmax[pl.ds(gg, 1), :] = m
            return carry

        lax.fori_loop(0, G, mx, 0)
        gmean = gsum / jnp.maximum(cnt, 1.0)
        emb = jnp.concatenate([gmean, gmax[...], gsum], axis=1)
        f1 = jnp.maximum(
            jnp.dot(emb, fw1_r[...], preferred_element_type=f32) + fb1_r[...], 0.0)
        f2 = jnp.maximum(
            jnp.dot(f1, fw2_r[...], preferred_element_type=f32) + fb2_r[...], 0.0)
        o_r[...] = jnp.dot(f2, cw_r[...], preferred_element_type=f32) + cb_r[...]

    full = lambda a: pl.BlockSpec(a.shape, lambda: tuple(0 for _ in a.shape))
    args = (h, batch, fw1, fb1.reshape(1, 2 * H), fw2, fb2.reshape(1, H),
            cw, cb.reshape(1, 4))
    return pl.pallas_call(
        body,
        in_specs=[full(a) for a in args],
        out_specs=full(jnp.zeros((G, 4))),
        out_shape=jax.ShapeDtypeStruct((G, 4), f32),
        scratch_shapes=[pltpu.VMEM((G, H), f32)],
    )(*args)


def kernel(x, edge_index, batch, node_emb, comp_emb, pin_emb, proj_W, proj_b,
           gin_W1, gin_b1, gin_W2, gin_b2, bn_g, bn_b,
           fus_W1, fus_b1, fus_W2, fus_b2, clf_W, clf_b):
    n = x.shape[0]
    xi = jnp.clip(x, 0, None).astype(jnp.int32)
    nt, ct, pt = xi[:, 0:1], xi[:, 1:2], xi[:, 2:3]
    e = edge_index.shape[1]
    nch = -(-e // (NW * EC))
    e_pad = NW * nch * EC
    rows_w = (-(-n // (NS * ZB))) * ZB
    n_pad = NS * rows_w
    if n_pad <= n:
        n_pad += ZB * NS
    src = edge_index[0].astype(jnp.int32)
    dst = edge_index[1].astype(jnp.int32)
    comb = src | (dst << 16)
    if e_pad > e:
        pad_ar = jnp.arange(e_pad - e, dtype=jnp.int32)
        comb = jnp.concatenate(
            [comb, (pad_ar % n) | ((n + pad_ar % (n_pad - n)) << 16)])
    comb3 = comb.reshape(NW, nch, EC)

    h = _embed_tc(nt, ct, pt, node_emb, comp_emb, pin_emb, proj_W, proj_b)
    for i in range(len(gin_W1)):
        parts = _edge_agg_sc(h, comb3, n_pad)
        h = _gin_tc(h, parts, gin_W1[i], gin_b1[i], gin_W2[i], gin_b2[i],
                    bn_g[i], bn_b[i], residual=(i > 0))
    return _pool_fuse_tc(h, batch.reshape(n, 1).astype(jnp.int32),
                         fus_W1, fus_b1, fus_W2, fus_b2, clf_W, clf_b)


# speedup vs baseline: 10.5899x; 1.2509x over previous
import functools

import jax
import jax.numpy as jnp
from jax import lax
from jax.experimental import pallas as pl
from jax.experimental.pallas import tpu as pltpu
from jax.experimental.pallas import tpu_sc as plsc

H = 128
G = 64
NC = 2
NS = 16
NW = NC * NS
EC = 96
ZB = 80


def _edge_agg_sc(h, comb3, n_pad):
    nch = comb3.shape[1]
    rows_w = n_pad // NS
    nzb = rows_w // ZB
    mesh = plsc.VectorSubcoreMesh(core_axis_name="c", subcore_axis_name="s")

    @functools.partial(
        pl.kernel,
        out_type=jax.ShapeDtypeStruct((NC, n_pad, H), jnp.float32),
        mesh=mesh,
        scratch_types=[
            pltpu.VMEM_SHARED((n_pad, H), jnp.float32),
            pltpu.VMEM((nch, EC), jnp.int32),
            pltpu.VMEM((EC,), jnp.int32),
            pltpu.VMEM((EC,), jnp.int32),
            pltpu.VMEM((EC,), jnp.int32),
            pltpu.VMEM((EC,), jnp.int32),
            pltpu.VMEM((EC, H), jnp.float32),
            pltpu.VMEM((EC, H), jnp.float32),
            pltpu.SemaphoreType.DMA,
            pltpu.SemaphoreType.DMA,
        ],
    )
    def k(h_hbm, comb_hbm, out_hbm, acc, comb, sb0, sb1, db0, db1,
          r0, r1, sem0, sem1):
        c = lax.axis_index("c")
        s = lax.axis_index("s")
        wid = c * NS + s

        icpy = pltpu.async_copy(comb_hbm.at[wid], comb, sem0)
        zero = jnp.zeros((16,), jnp.float32)

        def zrow(i, carry):
            for j in range(H // 16):
                r0[i, pl.ds(j * 16, 16)] = zero
            return carry

        lax.fori_loop(0, ZB, zrow, 0)
        base = s * rows_w
        for kk in range(nzb):
            pltpu.sync_copy(r0.at[pl.ds(0, ZB)],
                            acc.at[pl.ds(base + kk * ZB, ZB)])
        icpy.wait()
        plsc.subcore_barrier()

        def prep(j, sb, db):
            for t in range(EC // 16):
                cw = comb[j, pl.ds(t * 16, 16)]
                sb[pl.ds(t * 16, 16)] = cw & 0xFFFF
                db[pl.ds(t * 16, 16)] = cw >> 16

        def gather(sb, r, sem):
            return pltpu.async_copy(h_hbm.at[sb], r, sem)

        def wait(sb, r, sem):
            pltpu.make_async_copy(h_hbm.at[sb], r, sem).wait()

        def scatter(r, db):
            pltpu.sync_copy(r, acc.at[db], add=True)

        prep(0, sb0, db0)
        gather(sb0, r0, sem0)

        def pair(i, carry):
            j = 2 * i
            prep(j + 1, sb1, db1)
            gather(sb1, r1, sem1)
            wait(sb0, r0, sem0)
            scatter(r0, db0)

            @pl.when(j + 2 < nch)
            def _():
                prep(j + 2, sb0, db0)
                gather(sb0, r0, sem0)

            wait(sb1, r1, sem1)
            scatter(r1, db1)
            return carry

        lax.fori_loop(0, (nch - 1) // 2, pair, 0)
        if nch % 2 == 0:
            prep(nch - 1, sb1, db1)
            gather(sb1, r1, sem1)
            wait(sb0, r0, sem0)
            scatter(r0, db0)
            wait(sb1, r1, sem1)
            scatter(r1, db1)
        else:
            wait(sb0, r0, sem0)
            scatter(r0, db0)
        plsc.subcore_barrier()

        sl = pl.ds(base, rows_w)
        pltpu.sync_copy(acc.at[sl], out_hbm.at[c].at[sl])

    return k(h, comb3)


def _embed_tc(xi, node_emb, comp_emb, pin_emb, proj_W, proj_b):
    n = xi.shape[0]
    blk = 1000
    nb = n // blk

    def body(x_r, ne_r, ce_r, pe_r, w_r, b_r, o_r):
        f32 = jnp.float32
        tab = jnp.concatenate([
            jnp.dot(ne_r[...], w_r[0:H, :], preferred_element_type=f32),
            jnp.dot(ce_r[...], w_r[H:2 * H, :], preferred_element_type=f32),
            jnp.dot(pe_r[...], w_r[2 * H:3 * H, :], preferred_element_type=f32),
        ], axis=0)
        oh = jnp.concatenate([
            (x_r[:, 0:1] == lax.broadcasted_iota(jnp.int32, (blk, 5), 1)).astype(f32),
            (x_r[:, 1:2] == lax.broadcasted_iota(jnp.int32, (blk, 6), 1)).astype(f32),
            (x_r[:, 2:3] == lax.broadcasted_iota(jnp.int32, (blk, 6), 1)).astype(f32),
        ], axis=1)
        o_r[...] = jnp.dot(oh, tab, preferred_element_type=f32) + b_r[...]

    full = lambda a: pl.BlockSpec(a.shape, lambda b: tuple(0 for _ in a.shape))
    return pl.pallas_call(
        body,
        grid=(nb,),
        in_specs=[pl.BlockSpec((blk, 3), lambda b: (b, 0)), full(node_emb),
                  full(comp_emb), full(pin_emb), full(proj_W),
                  pl.BlockSpec((1, H), lambda b: (0, 0))],
        out_specs=pl.BlockSpec((blk, H), lambda b: (b, 0)),
        out_shape=jax.ShapeDtypeStruct((n, H), jnp.float32),
    )(xi, node_emb, comp_emb, pin_emb, proj_W, proj_b.reshape(1, H))


def _gin_tc(h, parts, w1, b1, w2, b2, g, bb, residual):
    n = h.shape[0]
    blk = 2000
    nb = n // blk
    f32 = jnp.float32

    def body(h_r, p_r, w1_r, b1_r, w2_r, b2_r, g_r, bb_r, o_r, vbuf, ssum, ssq):
        ph = pl.program_id(0)
        b = pl.program_id(1)

        @pl.when(jnp.logical_and(ph == 0, b == 0))
        def _():
            ssum[...] = jnp.zeros_like(ssum)
            ssq[...] = jnp.zeros_like(ssq)

        @pl.when(ph == 0)
        def _():
            z = h_r[...] + p_r[0] + p_r[1]
            u = jnp.maximum(
                jnp.dot(z, w1_r[...], preferred_element_type=f32) + b1_r[...], 0.0)
            v = jnp.dot(u, w2_r[...], preferred_element_type=f32) + b2_r[...]
            vbuf[pl.ds(b * blk, blk), :] = v
            ssum[...] += jnp.sum(v, axis=0, keepdims=True)
            ssq[...] += jnp.sum(v * v, axis=0, keepdims=True)

        @pl.when(ph == 1)
        def _():
            mean = ssum[...] * (1.0 / n)
            var = ssq[...] * (1.0 / n) - mean * mean
            inv = lax.rsqrt(var + 1e-5)
            v = vbuf[pl.ds(b * blk, blk), :]
            zz = jnp.maximum((v - mean) * inv * g_r[...] + bb_r[...], 0.0)
            if residual:
                zz = zz + h_r[...]
            o_r[...] = zz

    rowblk = pl.BlockSpec((blk, H), lambda p, b: (b, 0))
    full = lambda a: pl.BlockSpec(a.shape, lambda p, b: tuple(0 for _ in a.shape))
    return pl.pallas_call(
        body,
        grid=(2, nb),
        in_specs=[rowblk, pl.BlockSpec((NC, blk, H), lambda p, b: (0, b, 0)),
                  full(w1), pl.BlockSpec((1, 2 * H), lambda p, b: (0, 0)),
                  full(w2), pl.BlockSpec((1, H), lambda p, b: (0, 0)),
                  pl.BlockSpec((1, H), lambda p, b: (0, 0)),
                  pl.BlockSpec((1, H), lambda p, b: (0, 0))],
        out_specs=rowblk,
        out_shape=jax.ShapeDtypeStruct((n, H), f32),
        scratch_shapes=[pltpu.VMEM((n, H), f32), pltpu.VMEM((1, H), f32),
                        pltpu.VMEM((1, H), f32)],
    )(h, parts, w1, b1.reshape(1, 2 * H), w2, b2.reshape(1, H),
      g.reshape(1, H), bb.reshape(1, H))


def _pool_fuse_tc(h, batch, batp, fw1, fb1, fw2, fb2, cw, cb):
    n = h.shape[0]
    f32 = jnp.float32
    RB = 125
    NB = n // RB
    W = 128

    def body(h_r, bat_r, batp_r, fw1_r, fb1_r, fw2_r, fb2_r, cw_r, cb_r,
             o_r, gmax):
        hv = h_r[...]
        bat = bat_r[...]
        batp = batp_r[...]
        mask = (bat == lax.broadcasted_iota(jnp.int32, (n, G), 1)).astype(f32)
        gsum = lax.dot_general(mask, hv, (((0,), (0,)), ((), ())),
                               preferred_element_type=f32)
        cnt = jnp.sum(mask, axis=0)[:, None]
        bm = jnp.max(hv.reshape(NB, RB, H), axis=1)
        ninf = jnp.float32(-jnp.inf)

        def mx(g, carry):
            s = jnp.sum((batp < g).astype(jnp.int32))
            e2 = jnp.sum((batp < g + 1).astype(jnp.int32))
            fe = ((s + RB - 1) // RB) * RB
            sc = jnp.minimum(s, n - W)
            rows = lax.broadcasted_iota(jnp.int32, (W, H), 0) + sc
            wf = h_r[pl.ds(sc, W), :]
            mf = jnp.logical_and(rows >= s, rows < jnp.minimum(e2, fe))
            m1 = jnp.max(jnp.where(mf, wf, ninf), axis=0, keepdims=True)
            bl = e2 // RB
            bi = lax.broadcasted_iota(jnp.int32, (NB, H), 0)
            mm = jnp.logical_and(bi >= fe // RB, bi < bl)
            m2 = jnp.max(jnp.where(mm, bm, ninf), axis=0, keepdims=True)
            bs = bl * RB
            bsc = jnp.minimum(bs, n - W)
            rowsb = lax.broadcasted_iota(jnp.int32, (W, H), 0) + bsc
            wb = h_r[pl.ds(bsc, W), :]
            mb = jnp.logical_and(rowsb >= jnp.maximum(bs, s), rowsb < e2)
            m3 = jnp.max(jnp.where(mb, wb, ninf), axis=0, keepdims=True)
            gmax[pl.ds(g, 1), :] = jnp.maximum(jnp.maximum(m1, m2), m3)
            return carry

        lax.fori_loop(0, G, mx, 0)
        gmean = gsum / jnp.maximum(cnt, 1.0)
        emb = jnp.concatenate([gmean, gmax[...], gsum], axis=1)
        f1 = jnp.maximum(
            jnp.dot(emb, fw1_r[...], preferred_element_type=f32) + fb1_r[...], 0.0)
        f2 = jnp.maximum(
            jnp.dot(f1, fw2_r[...], preferred_element_type=f32) + fb2_r[...], 0.0)
        o_r[...] = jnp.dot(f2, cw_r[...], preferred_element_type=f32) + cb_r[...]

    full = lambda a: pl.BlockSpec(a.shape, lambda: tuple(0 for _ in a.shape))
    args = (h, batch, batp, fw1, fb1.reshape(1, 2 * H), fw2,
            fb2.reshape(1, H), cw, cb.reshape(1, 4))
    return pl.pallas_call(
        body,
        in_specs=[full(a) for a in args],
        out_specs=full(jnp.zeros((G, 4))),
        out_shape=jax.ShapeDtypeStruct((G, 4), f32),
        scratch_shapes=[pltpu.VMEM((G, H), f32)],
    )(*args)


def kernel(x, edge_index, batch, node_emb, comp_emb, pin_emb, proj_W, proj_b,
           gin_W1, gin_b1, gin_W2, gin_b2, bn_g, bn_b,
           fus_W1, fus_b1, fus_W2, fus_b2, clf_W, clf_b):
    n = x.shape[0]
    xi = jnp.clip(x, 0, None).astype(jnp.int32)
    e = edge_index.shape[1]
    nch = -(-e // (NW * EC))
    e_pad = NW * nch * EC
    rows_w = (-(-n // (NS * ZB))) * ZB
    n_pad = NS * rows_w
    if n_pad <= n:
        n_pad += ZB * NS
    src = edge_index[0].astype(jnp.int32)
    dst = edge_index[1].astype(jnp.int32)
    comb = src | (dst << 16)
    if e_pad > e:
        pad_ar = jnp.arange(e_pad - e, dtype=jnp.int32)
        comb = jnp.concatenate(
            [comb, (pad_ar % n) | ((n + pad_ar % (n_pad - n)) << 16)])
    comb3 = comb.reshape(NW, nch, EC)

    h = _embed_tc(xi, node_emb, comp_emb, pin_emb, proj_W, proj_b)
    for i in range(len(gin_W1)):
        parts = _edge_agg_sc(h, comb3, n_pad)
        h = _gin_tc(h, parts, gin_W1[i], gin_b1[i], gin_W2[i], gin_b2[i],
                    bn_g[i], bn_b[i], residual=(i > 0))
    bat32 = batch.astype(jnp.int32)
    npd = -(-n // H) * H
    batp = jnp.concatenate(
        [bat32, jnp.full((npd - n,), G, jnp.int32)]).reshape(npd // H, H)
    return _pool_fuse_tc(h, bat32.reshape(n, 1), batp,
                         fus_W1, fus_b1, fus_W2, fus_b2, clf_W, clf_b)
